# baseline (speedup 1.0000x reference)
import jax
import jax.numpy as jnp
from jax.experimental import pallas as pl
from jax.experimental.pallas import tpu as pltpu


def _diag_kernel(adj_ref, out_ref):
    out_ref[...] = jnp.broadcast_to(
        jnp.sum(adj_ref[...], axis=1, keepdims=True), out_ref.shape
    )


def kernel(input, adj, W):
    n = adj.shape[0]
    adj3 = adj.reshape(25, (n * n) // (25 * 128), 128)
    rows = adj3.shape[1]
    return pl.pallas_call(
        _diag_kernel,
        grid=(25,),
        in_specs=[pl.BlockSpec((1, rows, 128), lambda i: (i, 0, 0))],
        out_specs=pl.BlockSpec((1, 8, 128), lambda i: (i, 0, 0)),
        out_shape=jax.ShapeDtypeStruct((25, 8, 128), jnp.float32),
    )(adj3)


# speedup vs baseline: 4.9825x; 4.9825x over previous
import jax
import jax.numpy as jnp
from jax.experimental import pallas as pl
from jax.experimental.pallas import tpu as pltpu

_N = 10000
_BM = 400
_TBM = 80
_CHUNKS = [(i * _BM, _BM) for i in range(24)] + [
    (9600 + j * _TBM, _TBM) for j in range(5)
]


def _gcn_kernel(w_ref, x_hbm, adj_hbm, out_hbm,
                xbuf, abuf, obuf, support, xsem, asem, osem):
    nc = len(_CHUNKS)

    def a_copy(i):
        st, sz = _CHUNKS[i]
        b = i % 2
        return pltpu.make_async_copy(
            adj_hbm.at[pl.ds(st, sz), :], abuf.at[b, pl.ds(0, sz), :],
            asem.at[b])

    def o_copy(i):
        st, sz = _CHUNKS[i]
        b = i % 2
        return pltpu.make_async_copy(
            obuf.at[b, pl.ds(0, sz), :], out_hbm.at[pl.ds(st, sz), :],
            osem.at[b])

    a_copy(0).start()
    a_copy(1).start()
    xc = pltpu.make_async_copy(x_hbm, xbuf, xsem)
    xc.start()
    xc.wait()
    support[...] = jax.lax.dot(
        xbuf[...].astype(jnp.bfloat16),
        w_ref[...].astype(jnp.bfloat16),
        preferred_element_type=jnp.float32,
    ).astype(jnp.bfloat16)

    for i in range(nc):
        st, sz = _CHUNKS[i]
        b = i % 2
        a_copy(i).wait()
        res = jax.lax.dot(
            abuf[b, pl.ds(0, sz), :].astype(jnp.bfloat16),
            support[...],
            preferred_element_type=jnp.float32,
        )
        if i + 2 < nc:
            a_copy(i + 2).start()
        if i >= 2:
            o_copy(i - 2).wait()
        obuf[b, pl.ds(0, sz), :] = res
        o_copy(i).start()

    o_copy(nc - 2).wait()
    o_copy(nc - 1).wait()


def kernel(input, adj, W):
    n, d_in = input.shape
    d_out = W.shape[1]
    return pl.pallas_call(
        _gcn_kernel,
        in_specs=[
            pl.BlockSpec(memory_space=pltpu.MemorySpace.VMEM),
            pl.BlockSpec(memory_space=pltpu.MemorySpace.HBM),
            pl.BlockSpec(memory_space=pltpu.MemorySpace.HBM),
        ],
        out_specs=pl.BlockSpec(memory_space=pltpu.MemorySpace.HBM),
        out_shape=jax.ShapeDtypeStruct((n, d_out), jnp.float32),
        scratch_shapes=[
            pltpu.VMEM((n, d_in), jnp.float32),
            pltpu.VMEM((2, _BM, n), jnp.float32),
            pltpu.VMEM((2, _BM, d_out), jnp.float32),
            pltpu.VMEM((n, d_out), jnp.bfloat16),
            pltpu.SemaphoreType.DMA,
            pltpu.SemaphoreType.DMA((2,)),
            pltpu.SemaphoreType.DMA((2,)),
        ],
    )(W, input, adj)


# speedup vs baseline: 5.6740x; 1.1388x over previous
import jax
import jax.numpy as jnp
from jax.experimental import pallas as pl
from jax.experimental.pallas import tpu as pltpu

_BM = 400
_BK = 1280


def _gcn_kernel(x_ref, w_ref, adj_ref, out_ref, support_ref):
    @pl.when(pl.program_id(0) == 0)
    def _():
        support_ref[...] = jax.lax.dot(
            x_ref[...].astype(jnp.bfloat16),
            w_ref[...].astype(jnp.bfloat16),
            preferred_element_type=jnp.float32,
        ).astype(jnp.bfloat16)

    n = adj_ref.shape[1]
    acc = None
    for k0 in range(0, n, _BK):
        kw = min(_BK, n - k0)
        part = jax.lax.dot(
            adj_ref[:, k0:k0 + kw].astype(jnp.bfloat16),
            support_ref[k0:k0 + kw, :],
            preferred_element_type=jnp.float32,
        )
        acc = part if acc is None else acc + part
    out_ref[...] = acc


def kernel(input, adj, W):
    n, d_in = input.shape
    d_out = W.shape[1]
    grid = (n // _BM,)
    return pl.pallas_call(
        _gcn_kernel,
        grid=grid,
        in_specs=[
            pl.BlockSpec((n, d_in), lambda i: (0, 0)),
            pl.BlockSpec((d_in, d_out), lambda i: (0, 0)),
            pl.BlockSpec((_BM, n), lambda i: (i, 0)),
        ],
        out_specs=pl.BlockSpec((_BM, d_out), lambda i: (i, 0)),
        out_shape=jax.ShapeDtypeStruct((n, d_out), jnp.float32),
        scratch_shapes=[pltpu.VMEM((n, d_out), jnp.bfloat16)],
    )(input, W, adj)


# speedup vs baseline: 5.6962x; 1.0039x over previous
import jax
import jax.numpy as jnp
from jax.experimental import pallas as pl
from jax.experimental.pallas import tpu as pltpu

_BM = 400
_TBM = 80
_NMAIN = 24
_SLOTS = 4


def _gcn_kernel(x_ref, w_ref, adj_ref, adj_any, out_ref, support_ref,
                tailbuf, tsem):
    i = pl.program_id(0)
    base = _BM * _NMAIN

    def tail_copy(j):
        slot = j % _SLOTS
        return pltpu.make_async_copy(
            adj_any.at[pl.ds(base + _TBM * j, _TBM), :],
            tailbuf.at[slot], tsem.at[slot])

    @pl.when(i == _NMAIN - 1)
    def _():
        for j in range(_SLOTS):
            tail_copy(j).start()

    @pl.when(i == 0)
    def _():
        support_ref[...] = jax.lax.dot(
            x_ref[...].astype(jnp.bfloat16),
            w_ref[...].astype(jnp.bfloat16),
            preferred_element_type=jnp.float32,
        ).astype(jnp.bfloat16)

    @pl.when(i < _NMAIN)
    def _():
        out_ref[...] = jax.lax.dot(
            adj_ref[...].astype(jnp.bfloat16),
            support_ref[...],
            preferred_element_type=jnp.float32,
        )

    @pl.when(i == _NMAIN)
    def _():
        n_tail = _BM // _TBM
        for j in range(n_tail):
            slot = j % _SLOTS
            tail_copy(j).wait()
            res = jax.lax.dot(
                tailbuf[slot].astype(jnp.bfloat16),
                support_ref[...],
                preferred_element_type=jnp.float32,
            )
            if j + _SLOTS < n_tail:
                tail_copy(j + _SLOTS).start()
            out_ref[pl.ds(_TBM * j, _TBM), :] = res


def kernel(input, adj, W):
    n, d_in = input.shape
    d_out = W.shape[1]
    grid = (_NMAIN + 1,)
    return pl.pallas_call(
        _gcn_kernel,
        grid=grid,
        in_specs=[
            pl.BlockSpec((n, d_in), lambda i: (0, 0)),
            pl.BlockSpec((d_in, d_out), lambda i: (0, 0)),
            pl.BlockSpec((_BM, n), lambda i: (jnp.minimum(i, _NMAIN - 1), 0)),
            pl.BlockSpec(memory_space=pltpu.MemorySpace.HBM),
        ],
        out_specs=pl.BlockSpec((_BM, d_out), lambda i: (i, 0)),
        out_shape=jax.ShapeDtypeStruct((n, d_out), jnp.float32),
        scratch_shapes=[
            pltpu.VMEM((n, d_out), jnp.bfloat16),
            pltpu.VMEM((_SLOTS, _TBM, n), jnp.float32),
            pltpu.SemaphoreType.DMA((_SLOTS,)),
        ],
        compiler_params=pltpu.CompilerParams(vmem_limit_bytes=67108864),
    )(input, W, adj, adj)
